# Initial kernel scaffold; baseline (speedup 1.0000x reference)
import jax
import jax.numpy as jnp
from jax.experimental import pallas as pl


def kernel(x, edge_index, W_feat, W_conv, W_attn, b_attn, q_attn, W_concat):
    raise NotImplementedError("write your pallas kernel here")


# speedup vs baseline: 2.8220x; 2.8220x over previous
import functools

import jax
import jax.numpy as jnp
from jax import lax
from jax.experimental import pallas as pl
from jax.experimental.pallas import tpu as pltpu
from jax.experimental.pallas import tpu_sc as plsc

N = 10000
R = 4
E = 80000
DF = 128
DH = 128
DE = 128
DQ = 64

NC = 2
NS = 16
WA = 144
NPAD = 10240
BN = 256
EPT = E // NS
CH = 128
NCHUNK = (EPT + CH - 1) // CH
EPAD = NCHUNK * CH - EPT
ZCH = NPAD // (NS * CH)


def _haug_body(x_ref, wf_ref, o_ref):
    h = jnp.tanh(
        lax.dot_general(x_ref[...], wf_ref[...], (((1,), (1,)), ((), ())),
                        preferred_element_type=jnp.float32))
    ones = jnp.ones((BN, 1), jnp.float32)
    zeros = jnp.zeros((BN, WA - DH - 1), jnp.float32)
    o_ref[...] = jnp.concatenate([h, ones, zeros], axis=1)


def _haug(x_pad, w_feat):
    return pl.pallas_call(
        _haug_body,
        grid=(NPAD // BN,),
        in_specs=[
            pl.BlockSpec((BN, DF), lambda i: (i, 0)),
            pl.BlockSpec((DH, DF), lambda i: (0, 0)),
        ],
        out_specs=pl.BlockSpec((BN, WA), lambda i: (i, 0)),
        out_shape=jax.ShapeDtypeStruct((NPAD, WA), jnp.float32),
    )(x_pad, w_feat)


def _seg_body(h_hbm, idx_hbm, zeros_hbm, out_hbm,
              rows, sidx, didx, acc, sem):
    c = lax.axis_index("c")
    s = lax.axis_index("s")
    for i in range(R // NC):
        r = c * (R // NC) + i
        pltpu.sync_copy(zeros_hbm, rows)
        for z in range(ZCH):
            row0 = (s * ZCH + z) * CH
            pltpu.sync_copy(rows, acc.at[pl.ds(row0, CH)])
        pltpu.sync_copy(idx_hbm.at[r, 0, s], sidx)
        pltpu.sync_copy(idx_hbm.at[r, 1, s], didx)
        plsc.subcore_barrier()

        def chunk(j, carry):
            pltpu.async_copy(h_hbm.at[sidx.at[j]], rows, sem).wait()
            pltpu.sync_copy(rows, acc.at[didx.at[j]], add=True)
            return carry

        lax.fori_loop(0, NCHUNK, chunk, 0)
        plsc.subcore_barrier()
        for z in range(ZCH):
            row0 = (s * ZCH + z) * CH
            pltpu.sync_copy(acc.at[pl.ds(row0, CH)],
                            out_hbm.at[r, pl.ds(row0, CH)])


def _segsum(h_aug, idxp, zeros_in):
    mesh = plsc.VectorSubcoreMesh(
        core_axis_name="c", subcore_axis_name="s",
        num_cores=NC, num_subcores=NS)
    f = pl.kernel(
        _seg_body,
        out_type=jax.ShapeDtypeStruct((R, NPAD, WA), jnp.float32),
        mesh=mesh,
        scratch_types=[
            pltpu.VMEM((CH, WA), jnp.float32),
            pltpu.VMEM((NCHUNK, CH), jnp.int32),
            pltpu.VMEM((NCHUNK, CH), jnp.int32),
            pltpu.VMEM_SHARED((NPAD, WA), jnp.float32),
            pltpu.SemaphoreType.DMA,
        ],
        compiler_params=pltpu.CompilerParams(use_tc_tiling_on_sc=False),
    )
    return f(h_aug, idxp, zeros_in)


def _final_body(s_ref, ha_ref, wconv_ref, wattn_ref, b_ref, q_ref, wcat_ref,
                y_ref, a_ref):
    hr = []
    scores = []
    for r in range(R):
        srow = s_ref[r]
        deg = jnp.maximum(srow[:, DH:DH + 1], 1.0)
        agg = srow[:, :DH] / deg
        h_r = lax.dot_general(agg, wconv_ref[r], (((1,), (1,)), ((), ())),
                              preferred_element_type=jnp.float32)
        t = jnp.tanh(
            lax.dot_general(h_r, wattn_ref[...], (((1,), (1,)), ((), ())),
                            preferred_element_type=jnp.float32)
            + b_ref[...])
        sc = jnp.sum(t * q_ref[...], axis=1, keepdims=True)
        hr.append(h_r)
        scores.append(sc)
    sc = jnp.concatenate(scores, axis=1)
    m = jnp.max(sc, axis=1, keepdims=True)
    ex = jnp.exp(sc - m)
    alpha = ex / jnp.sum(ex, axis=1, keepdims=True)
    h1 = alpha[:, 0:1] * hr[0]
    for r in range(1, R):
        h1 = h1 + alpha[:, r:r + 1] * hr[r]
    h0 = ha_ref[:, :DH]
    wcat = wcat_ref[...]
    y = (lax.dot_general(h0, wcat[:, :DH], (((1,), (1,)), ((), ())),
                         preferred_element_type=jnp.float32)
         + lax.dot_general(h1, wcat[:, DH:], (((1,), (1,)), ((), ())),
                           preferred_element_type=jnp.float32))
    y_ref[...] = y
    a_ref[...] = alpha


def _final(sums, h_aug, w_conv, w_attn, b_attn, q_attn, w_cat):
    return pl.pallas_call(
        _final_body,
        grid=(NPAD // BN,),
        in_specs=[
            pl.BlockSpec((R, BN, WA), lambda i: (0, i, 0)),
            pl.BlockSpec((BN, WA), lambda i: (i, 0)),
            pl.BlockSpec((R, DE, DH), lambda i: (0, 0, 0)),
            pl.BlockSpec((DQ, DE), lambda i: (0, 0)),
            pl.BlockSpec((1, DQ), lambda i: (0, 0)),
            pl.BlockSpec((1, DQ), lambda i: (0, 0)),
            pl.BlockSpec((DE, 2 * DE), lambda i: (0, 0)),
        ],
        out_specs=[
            pl.BlockSpec((BN, DE), lambda i: (i, 0)),
            pl.BlockSpec((BN, R), lambda i: (i, 0)),
        ],
        out_shape=[
            jax.ShapeDtypeStruct((NPAD, DE), jnp.float32),
            jax.ShapeDtypeStruct((NPAD, R), jnp.float32),
        ],
    )(sums, h_aug, w_conv, w_attn, b_attn, q_attn, w_cat)


def kernel(x, edge_index, W_feat, W_conv, W_attn, b_attn, q_attn, W_concat):
    x_pad = jnp.zeros((NPAD, DF), jnp.float32).at[:N].set(x)
    idx = edge_index.reshape(R, 2, NS, EPT)
    pad = jnp.full((R, 2, NS, EPAD), N, dtype=jnp.int32)
    idxp = jnp.concatenate([idx, pad], axis=3).reshape(R, 2, NS, NCHUNK, CH)
    zeros_in = jnp.zeros((CH, WA), jnp.float32)

    h_aug = _haug(x_pad, W_feat)
    sums = _segsum(h_aug, idxp, zeros_in)
    y, attn = _final(sums, h_aug, W_conv, W_attn,
                     b_attn.reshape(1, DQ), q_attn.reshape(1, DQ), W_concat)
    return (y[:N], attn[:N])
